# baseline (speedup 1.0000x reference)
import functools

import jax
import jax.numpy as jnp
from jax import lax
from jax.experimental import pallas as pl
from jax.experimental.pallas import tpu as pltpu
from jax.experimental.pallas import tpu_sc as plsc

NUM_CODEBOOKS = 32
CODEBOOK_VOCAB_SIZE = 2051
HIDDEN = 2048

NC = 2
NS = 16
NW = NC * NS
LANES = 16

N_TOKENS = 4 * 8192
B_PER_W = N_TOKENS // NW
CH = 16
NCH = B_PER_W // CH
NBUF = 3
NCH_MAIN = NCH - (NCH % NBUF)


def _make_kernel():
    mesh = plsc.VectorSubcoreMesh(core_axis_name="c", subcore_axis_name="s")

    @functools.partial(
        pl.kernel,
        out_type=jax.ShapeDtypeStruct((N_TOKENS, HIDDEN), jnp.float32),
        mesh=mesh,
        scratch_types=[
            pltpu.VMEM((B_PER_W,), jnp.int32),
            pltpu.VMEM((B_PER_W,), jnp.int32),
            pltpu.VMEM((NCH, CH), jnp.int32),
            pltpu.VMEM((NBUF, CH, HIDDEN), jnp.float32),
        ] + [pltpu.SemaphoreType.DMA] * (2 * NBUF),
    )
    def embed(ids_hbm, cb_hbm, table_hbm, out_hbm,
              ids_v, cb_v, idx_v, rows_v, *sems):
        gsem = sems[:NBUF]
        osem = sems[NBUF:]
        wid = lax.axis_index("s") * NC + lax.axis_index("c")
        base = wid * B_PER_W

        pltpu.sync_copy(ids_hbm.at[pl.ds(base, B_PER_W)], ids_v)
        pltpu.sync_copy(cb_hbm.at[pl.ds(base, B_PER_W)], cb_v)

        for i in range(NCH):
            idx_v[i, :] = (ids_v[pl.ds(i * CH, CH)]
                           + cb_v[pl.ds(i * CH, CH)] * CODEBOOK_VOCAB_SIZE)

        for b in range(NBUF):
            pltpu.async_copy(table_hbm.at[idx_v.at[b]], rows_v.at[b], gsem[b])

        @pl.loop(0, NCH_MAIN, step=NBUF)
        def _(c0):
            for b in range(NBUF):
                c = c0 + b
                pltpu.make_async_copy(
                    table_hbm.at[idx_v.at[b]], rows_v.at[b], gsem[b]).wait()
                out_slice = out_hbm.at[pl.ds(base + c * CH, CH)]
                odesc = pltpu.async_copy(rows_v.at[b], out_slice, osem[b])
                odesc.wait()
                nxt = c + NBUF

                @pl.when(nxt < NCH)
                def _():
                    pltpu.async_copy(
                        table_hbm.at[idx_v.at[nxt]], rows_v.at[b], gsem[b])

        for c in range(NCH_MAIN, NCH):
            b = c % NBUF
            pltpu.make_async_copy(
                table_hbm.at[idx_v.at[b]], rows_v.at[b], gsem[b]).wait()
            out_slice = out_hbm.at[pl.ds(base + c * CH, CH)]
            pltpu.async_copy(rows_v.at[b], out_slice, osem[b]).wait()

    return embed


_embed = _make_kernel()


def kernel(input_ids, codebook_idxs, table):
    ids = input_ids.reshape(-1)
    cb = codebook_idxs.reshape(-1)
    out = _embed(ids, cb, table)
    return out.reshape(*input_ids.shape, HIDDEN)


# speedup vs baseline: 1.0082x; 1.0082x over previous
import functools

import jax
import jax.numpy as jnp
from jax import lax
from jax.experimental import pallas as pl
from jax.experimental.pallas import tpu as pltpu
from jax.experimental.pallas import tpu_sc as plsc

NUM_CODEBOOKS = 32
CODEBOOK_VOCAB_SIZE = 2051
HIDDEN = 2048

NC = 2
NS = 16
NW = NC * NS
LANES = 16

N_TOKENS = 4 * 8192
B_PER_W = N_TOKENS // NW
CH = 16
NCH = B_PER_W // CH
NBUF = 3
NCH_MAIN = NCH - (NCH % NBUF)


def _make_kernel():
    mesh = plsc.VectorSubcoreMesh(core_axis_name="c", subcore_axis_name="s")

    @functools.partial(
        pl.kernel,
        out_type=jax.ShapeDtypeStruct((N_TOKENS, HIDDEN), jnp.float32),
        mesh=mesh,
        scratch_types=[
            pltpu.VMEM((B_PER_W,), jnp.int32),
            pltpu.VMEM((B_PER_W,), jnp.int32),
            pltpu.VMEM((NCH, CH), jnp.int32),
            pltpu.VMEM((NBUF, CH, HIDDEN), jnp.float32),
        ] + [pltpu.SemaphoreType.DMA] * (2 * NBUF),
    )
    def embed(ids_hbm, cb_hbm, table_hbm, out_hbm,
              ids_v, cb_v, idx_v, rows_v, *sems):
        gsem = sems[:NBUF]
        osem = sems[NBUF:]
        wid = lax.axis_index("s") * NC + lax.axis_index("c")
        base = wid * B_PER_W

        head = NBUF * CH
        pltpu.sync_copy(ids_hbm.at[pl.ds(base, head)], ids_v.at[pl.ds(0, head)])
        pltpu.sync_copy(cb_hbm.at[pl.ds(base, head)], cb_v.at[pl.ds(0, head)])
        for i in range(NBUF):
            idx_v[i, :] = (ids_v[pl.ds(i * CH, CH)]
                           + cb_v[pl.ds(i * CH, CH)] * CODEBOOK_VOCAB_SIZE)

        for b in range(NBUF):
            pltpu.async_copy(table_hbm.at[idx_v.at[b]], rows_v.at[b], gsem[b])

        pltpu.sync_copy(ids_hbm.at[pl.ds(base + head, B_PER_W - head)],
                        ids_v.at[pl.ds(head, B_PER_W - head)])
        pltpu.sync_copy(cb_hbm.at[pl.ds(base + head, B_PER_W - head)],
                        cb_v.at[pl.ds(head, B_PER_W - head)])
        for i in range(NBUF, NCH):
            idx_v[i, :] = (ids_v[pl.ds(i * CH, CH)]
                           + cb_v[pl.ds(i * CH, CH)] * CODEBOOK_VOCAB_SIZE)

        @pl.loop(0, NCH_MAIN, step=NBUF)
        def _(c0):
            for b in range(NBUF):
                c = c0 + b
                pltpu.make_async_copy(
                    table_hbm.at[idx_v.at[b]], rows_v.at[b], gsem[b]).wait()
                out_slice = out_hbm.at[pl.ds(base + c * CH, CH)]
                odesc = pltpu.async_copy(rows_v.at[b], out_slice, osem[b])
                odesc.wait()
                nxt = c + NBUF

                @pl.when(nxt < NCH)
                def _():
                    pltpu.async_copy(
                        table_hbm.at[idx_v.at[nxt]], rows_v.at[b], gsem[b])

        for c in range(NCH_MAIN, NCH):
            b = c % NBUF
            pltpu.make_async_copy(
                table_hbm.at[idx_v.at[b]], rows_v.at[b], gsem[b]).wait()
            out_slice = out_hbm.at[pl.ds(base + c * CH, CH)]
            pltpu.async_copy(rows_v.at[b], out_slice, osem[b]).wait()

    return embed


_embed = _make_kernel()


def kernel(input_ids, codebook_idxs, table):
    ids = input_ids.reshape(-1)
    cb = codebook_idxs.reshape(-1)
    out = _embed(ids, cb, table)
    return out.reshape(*input_ids.shape, HIDDEN)
